# baseline (speedup 1.0000x reference)
import functools

import jax
import jax.numpy as jnp
from jax import lax
from jax.experimental import pallas as pl
from jax.experimental.pallas import tpu as pltpu
from jax.experimental.pallas import tpu_sc as plsc

N0, N1, N2 = 50000, 12500, 3125
N0P, N1P, N2P = 50176, 12544, 3200
NW = 32


def _sc_gather(table, idx3d, b_pad, d, n_chunks, chunk, tc_tiling=False):
    b_per_w = n_chunks * chunk
    mesh = plsc.VectorSubcoreMesh(core_axis_name="c", subcore_axis_name="s")

    @functools.partial(
        pl.kernel,
        mesh=mesh,
        out_type=jax.ShapeDtypeStruct((b_pad, d), jnp.float32),
        compiler_params=pltpu.CompilerParams(use_tc_tiling_on_sc=tc_tiling),
        scratch_types=[
            pltpu.VMEM((n_chunks, chunk), jnp.int32),
            pltpu.VMEM((b_per_w, d), jnp.float32),
            pltpu.SemaphoreType.DMA,
        ],
    )
    def k(table_hbm, idx_hbm, out_hbm, idx_v, rows_v, sem):
        wid = lax.axis_index("s") * 2 + lax.axis_index("c")
        base = wid * b_per_w
        pltpu.sync_copy(idx_hbm.at[wid], idx_v)
        copies = []
        for j in range(n_chunks):
            copies.append(
                pltpu.async_copy(
                    table_hbm.at[idx_v.at[j]],
                    rows_v.at[pl.ds(j * chunk, chunk)],
                    sem,
                )
            )
        for c in copies:
            c.wait()
        pltpu.sync_copy(rows_v, out_hbm.at[pl.ds(base, b_per_w)])

    return k(table, idx3d)


def _sc_gather_stream(table, idx3d, b_pad, d, n_chunks, chunk, nbuf=4):
    b_per_w = n_chunks * chunk
    mesh = plsc.VectorSubcoreMesh(core_axis_name="c", subcore_axis_name="s")

    @functools.partial(
        pl.kernel,
        mesh=mesh,
        out_type=jax.ShapeDtypeStruct((b_pad, d), jnp.float32),
        compiler_params=pltpu.CompilerParams(use_tc_tiling_on_sc=True),
        scratch_types=[
            pltpu.VMEM((n_chunks, chunk), jnp.int32),
            pltpu.VMEM((nbuf, chunk, d), jnp.float32),
            pltpu.SemaphoreType.DMA,
            pltpu.SemaphoreType.DMA,
        ],
    )
    def k(table_hbm, idx_hbm, out_hbm, idx_v, rows_v, gsem, wsem):
        wid = lax.axis_index("s") * 2 + lax.axis_index("c")
        base = wid * b_per_w
        pltpu.sync_copy(idx_hbm.at[wid], idx_v)
        gathers = []
        for j in range(min(nbuf, n_chunks)):
            gathers.append(
                pltpu.async_copy(table_hbm.at[idx_v.at[j]], rows_v.at[j % nbuf], gsem)
            )
        writes = [None] * n_chunks
        for j in range(n_chunks):
            gathers[j].wait()
            writes[j] = pltpu.async_copy(
                rows_v.at[j % nbuf],
                out_hbm.at[pl.ds(base + j * chunk, chunk)],
                wsem,
            )
            nxt = j + nbuf
            if nxt < n_chunks:
                writes[j].wait()
                gathers.append(
                    pltpu.async_copy(table_hbm.at[idx_v.at[nxt]], rows_v.at[nxt % nbuf], gsem)
                )
        for j in range(max(0, n_chunks - nbuf), n_chunks):
            writes[j].wait()

    return k(table, idx3d)


def _tc_coarse(x2, w1a):
    def body(f_ref, w_ref, o_ref):
        o_ref[...] = jnp.dot(
            f_ref[...].astype(jnp.bfloat16), w_ref[...].astype(jnp.bfloat16),
            preferred_element_type=jnp.float32)

    return pl.pallas_call(
        body,
        grid=(1,),
        in_specs=[
            pl.BlockSpec((N2P, 256), lambda i: (0, 0)),
            pl.BlockSpec((256, 128), lambda i: (0, 0)),
        ],
        out_specs=pl.BlockSpec((N2P, 128), lambda i: (0, 0)),
        out_shape=jax.ShapeDtypeStruct((N2P, 128), jnp.float32),
    )(x2, w1a)


def _tc_skip1(skip1, w1b):
    def body(s_ref, w_ref, o_ref):
        o_ref[...] = jnp.dot(
            s_ref[...].astype(jnp.bfloat16), w_ref[...].astype(jnp.bfloat16),
            preferred_element_type=jnp.float32)

    return pl.pallas_call(
        body,
        grid=(1,),
        in_specs=[
            pl.BlockSpec((N1P, 256), lambda i: (0, 0)),
            pl.BlockSpec((256, 128), lambda i: (0, 0)),
        ],
        out_specs=pl.BlockSpec((N1P, 128), lambda i: (0, 0)),
        out_shape=jax.ShapeDtypeStruct((N1P, 128), jnp.float32),
    )(skip1, w1b)


def _tc_mid(g1, s1, gamma1, beta1, w2a):
    def body(g_ref, s_ref, gm_ref, bt_ref, w2a_ref, o_ref):
        h = g_ref[...] + s_ref[...]
        rows = lax.broadcasted_iota(jnp.int32, (N1P, 128), 0)
        mask = rows < N1
        hm = jnp.where(mask, h, 0.0)
        mean = jnp.sum(hm, axis=0, keepdims=True) / N1
        var = jnp.sum(jnp.where(mask, h * h, 0.0), axis=0, keepdims=True) / N1 - mean * mean
        scale = lax.rsqrt(var + 1e-5) * gm_ref[...]
        hn = (h - mean) * scale + bt_ref[...]
        act = jnp.where(hn >= 0.0, hn, 0.1 * hn)
        o_ref[...] = jnp.dot(
            act.astype(jnp.bfloat16), w2a_ref[...].astype(jnp.bfloat16),
            preferred_element_type=jnp.float32)

    return pl.pallas_call(
        body,
        out_shape=jax.ShapeDtypeStruct((N1P, 128), jnp.float32),
    )(g1, s1, gamma1.reshape(1, 128), beta1.reshape(1, 128),
      jnp.pad(w2a, ((0, 0), (0, 64))))


def _tc_skip0(skip0, w2b, b2):
    R = 6272
    nb = (N0 + R - 1) // R

    def body(s_ref, w_ref, b_ref, o_ref):
        o_ref[...] = (
            lax.dot_general(
                w_ref[...].astype(jnp.bfloat16), s_ref[...].astype(jnp.bfloat16),
                (((0,), (1,)), ((), ())),
                preferred_element_type=jnp.float32,
            )
            + b_ref[...]
        )

    return pl.pallas_call(
        body,
        grid=(nb,),
        in_specs=[
            pl.BlockSpec((R, 128), lambda i: (i, 0)),
            pl.BlockSpec((128, 64), lambda i: (0, 0)),
            pl.BlockSpec((64, 1), lambda i: (0, 0)),
        ],
        out_specs=pl.BlockSpec((64, R), lambda i: (0, i)),
        out_shape=jax.ShapeDtypeStruct((64, N0), jnp.float32),
    )(skip0, w2b, b2.reshape(64, 1))


def _tc_add(g0, s0_t):
    R = 6272
    nb = (N0 + R - 1) // R

    def body(g_ref, s_ref, o_ref):
        o_ref[...] = jnp.transpose(g_ref[:, :64]) + s_ref[...]

    return pl.pallas_call(
        body,
        grid=(nb,),
        in_specs=[
            pl.BlockSpec((R, 128), lambda i: (i, 0)),
            pl.BlockSpec((64, R), lambda i: (0, i)),
        ],
        out_specs=pl.BlockSpec((64, R), lambda i: (0, i)),
        out_shape=jax.ShapeDtypeStruct((64, N0), jnp.float32),
    )(g0, s0_t)


def _pad_idx(idx, n_pad, n_chunks, chunk):
    flat = jnp.pad(idx[:, 0].astype(jnp.int32), (0, n_pad - idx.shape[0]))
    return flat.reshape(NW, n_chunks, chunk)


def kernel(feats0, skip0, skip1, up0, up1, W1, b1, gamma1, beta1, W2, b2):
    del b1
    w1a, w1b = W1[:256], W1[256:]
    w2a, w2b = W2[:128], W2[128:]

    idx1 = _pad_idx(up1, N1P, 7, 56)
    idx0 = _pad_idx(up0, N0P, 14, 112)

    z2 = _tc_coarse(feats0.T, w1a)
    s1 = _tc_skip1(skip1, w1b)
    g1 = _sc_gather(z2, idx1, N1P, 128, 7, 56, tc_tiling=True)
    y = _tc_mid(g1, s1, gamma1, beta1, w2a)
    s0_t = _tc_skip0(skip0, w2b, b2)
    g0 = _sc_gather_stream(y, idx0, N0P, 128, 14, 112, nbuf=8)
    return _tc_add(g0, s0_t).T


# speedup vs baseline: 1.0401x; 1.0401x over previous
import functools

import jax
import jax.numpy as jnp
from jax import lax
from jax.experimental import pallas as pl
from jax.experimental.pallas import tpu as pltpu
from jax.experimental.pallas import tpu_sc as plsc

N0, N1, N2 = 50000, 12500, 3125
N0P, N1P, N2P = 50176, 12544, 3200
NW = 32


def _sc_gather(table, idx3d, b_pad, d, n_chunks, chunk, table_rows):
    b_per_w = n_chunks * chunk
    rpt = table_rows // 16
    mesh = plsc.VectorSubcoreMesh(core_axis_name="c", subcore_axis_name="s")

    @functools.partial(
        pl.kernel,
        mesh=mesh,
        out_type=jax.ShapeDtypeStruct((b_pad, d), jnp.float32),
        compiler_params=pltpu.CompilerParams(use_tc_tiling_on_sc=True),
        scratch_types=[
            pltpu.VMEM((n_chunks, chunk), jnp.int32),
            pltpu.VMEM((b_per_w, d), jnp.float32),
            pltpu.VMEM_SHARED((table_rows, d), jnp.float32),
            pltpu.SemaphoreType.DMA,
        ],
    )
    def k(table_hbm, idx_hbm, out_hbm, idx_v, rows_v, table_sp, sem):
        c = lax.axis_index("c")
        s = lax.axis_index("s")
        wid = s * 2 + c
        base = wid * b_per_w
        pltpu.sync_copy(
            table_hbm.at[pl.ds(s * rpt, rpt)], table_sp.at[pl.ds(s * rpt, rpt)]
        )
        pltpu.sync_copy(idx_hbm.at[wid], idx_v)
        plsc.subcore_barrier()
        copies = []
        for j in range(n_chunks):
            copies.append(
                pltpu.async_copy(
                    table_sp.at[idx_v.at[j]],
                    rows_v.at[pl.ds(j * chunk, chunk)],
                    sem,
                )
            )
        for cp in copies:
            cp.wait()
        pltpu.sync_copy(rows_v, out_hbm.at[pl.ds(base, b_per_w)])

    return k(table, idx3d)


def _sc_gather_stream(table, idx3d, b_pad, d, n_chunks, chunk, nbuf=4):
    b_per_w = n_chunks * chunk
    mesh = plsc.VectorSubcoreMesh(core_axis_name="c", subcore_axis_name="s")

    @functools.partial(
        pl.kernel,
        mesh=mesh,
        out_type=jax.ShapeDtypeStruct((b_pad, d), jnp.float32),
        compiler_params=pltpu.CompilerParams(use_tc_tiling_on_sc=True),
        scratch_types=[
            pltpu.VMEM((n_chunks, chunk), jnp.int32),
            pltpu.VMEM((nbuf, chunk, d), jnp.float32),
            pltpu.SemaphoreType.DMA,
            pltpu.SemaphoreType.DMA,
        ],
    )
    def k(table_hbm, idx_hbm, out_hbm, idx_v, rows_v, gsem, wsem):
        wid = lax.axis_index("s") * 2 + lax.axis_index("c")
        base = wid * b_per_w
        pltpu.sync_copy(idx_hbm.at[wid], idx_v)
        gathers = []
        for j in range(min(nbuf, n_chunks)):
            gathers.append(
                pltpu.async_copy(table_hbm.at[idx_v.at[j]], rows_v.at[j % nbuf], gsem)
            )
        writes = [None] * n_chunks
        for j in range(n_chunks):
            gathers[j].wait()
            writes[j] = pltpu.async_copy(
                rows_v.at[j % nbuf],
                out_hbm.at[pl.ds(base + j * chunk, chunk)],
                wsem,
            )
            nxt = j + nbuf
            if nxt < n_chunks:
                writes[j].wait()
                gathers.append(
                    pltpu.async_copy(table_hbm.at[idx_v.at[nxt]], rows_v.at[nxt % nbuf], gsem)
                )
        for j in range(max(0, n_chunks - nbuf), n_chunks):
            writes[j].wait()

    return k(table, idx3d)


def _tc_coarse(x2, w1a):
    def body(f_ref, w_ref, o_ref):
        o_ref[...] = jnp.dot(
            f_ref[...].astype(jnp.bfloat16), w_ref[...].astype(jnp.bfloat16),
            preferred_element_type=jnp.float32)

    return pl.pallas_call(
        body,
        grid=(1,),
        in_specs=[
            pl.BlockSpec((N2P, 256), lambda i: (0, 0)),
            pl.BlockSpec((256, 128), lambda i: (0, 0)),
        ],
        out_specs=pl.BlockSpec((N2P, 128), lambda i: (0, 0)),
        out_shape=jax.ShapeDtypeStruct((N2P, 128), jnp.float32),
    )(x2, w1a)


def _tc_skip1(skip1, w1b):
    def body(s_ref, w_ref, o_ref):
        o_ref[...] = jnp.dot(
            s_ref[...].astype(jnp.bfloat16), w_ref[...].astype(jnp.bfloat16),
            preferred_element_type=jnp.float32)

    R = 3136
    return pl.pallas_call(
        body,
        grid=(N1P // R,),
        in_specs=[
            pl.BlockSpec((R, 256), lambda i: (i, 0)),
            pl.BlockSpec((256, 128), lambda i: (1, 0)),
        ],
        out_specs=pl.BlockSpec((R, 128), lambda i: (i, 0)),
        out_shape=jax.ShapeDtypeStruct((N1P, 128), jnp.float32),
        compiler_params=pltpu.CompilerParams(dimension_semantics=("arbitrary",)),
    )(skip1, w1b)


def _tc_mid(g1, s1, gamma1, beta1, w2a):
    def body(g_ref, s_ref, gm_ref, bt_ref, w2a_ref, o_ref):
        h = g_ref[...] + s_ref[...]
        rows = lax.broadcasted_iota(jnp.int32, (N1P, 128), 0)
        mask = rows < N1
        hm = jnp.where(mask, h, 0.0)
        mean = jnp.sum(hm, axis=0, keepdims=True) / N1
        var = jnp.sum(jnp.where(mask, h * h, 0.0), axis=0, keepdims=True) / N1 - mean * mean
        scale = lax.rsqrt(var + 1e-5) * gm_ref[...]
        hn = (h - mean) * scale + bt_ref[...]
        act = jnp.where(hn >= 0.0, hn, 0.1 * hn)
        o_ref[...] = jnp.dot(
            act.astype(jnp.bfloat16), w2a_ref[...].astype(jnp.bfloat16),
            preferred_element_type=jnp.float32)

    return pl.pallas_call(
        body,
        out_shape=jax.ShapeDtypeStruct((N1P, 128), jnp.float32),
    )(g1, s1, gamma1.reshape(1, 128), beta1.reshape(1, 128),
      jnp.pad(w2a, ((0, 0), (0, 64))))


def _tc_skip0(skip0, w2b, b2):
    R = 6272
    nb = (N0 + R - 1) // R

    def body(s_ref, w_ref, b_ref, o_ref):
        o_ref[...] = (
            lax.dot_general(
                w_ref[...].astype(jnp.bfloat16), s_ref[...].astype(jnp.bfloat16),
                (((0,), (1,)), ((), ())),
                preferred_element_type=jnp.float32,
            )
            + b_ref[...]
        )

    return pl.pallas_call(
        body,
        grid=(nb,),
        in_specs=[
            pl.BlockSpec((R, 128), lambda i: (i, 0)),
            pl.BlockSpec((128, 64), lambda i: (1, 0)),
            pl.BlockSpec((64, 1), lambda i: (0, 0)),
        ],
        out_specs=pl.BlockSpec((64, R), lambda i: (0, i)),
        out_shape=jax.ShapeDtypeStruct((64, N0), jnp.float32),
    )(skip0, w2b, b2.reshape(64, 1))


def _tc_add(g0, s0_t):
    R = 6272
    nb = (N0 + R - 1) // R

    def body(g_ref, s_ref, o_ref):
        o_ref[...] = jnp.transpose(g_ref[:, :64]) + s_ref[...]

    return pl.pallas_call(
        body,
        grid=(nb,),
        in_specs=[
            pl.BlockSpec((R, 128), lambda i: (i, 0)),
            pl.BlockSpec((64, R), lambda i: (0, i)),
        ],
        out_specs=pl.BlockSpec((64, R), lambda i: (0, i)),
        out_shape=jax.ShapeDtypeStruct((64, N0), jnp.float32),
    )(g0, s0_t)


def _pad_idx(idx, n_pad, n_chunks, chunk):
    flat = jnp.pad(idx[:, 0].astype(jnp.int32), (0, n_pad - idx.shape[0]))
    return flat.reshape(NW, n_chunks, chunk)


def kernel(feats0, skip0, skip1, up0, up1, W1, b1, gamma1, beta1, W2, b2):
    del b1
    w2a = W2[:128]

    idx1 = _pad_idx(up1, N1P, 7, 56)
    idx0 = _pad_idx(up0, N0P, 14, 112)

    z2 = _tc_coarse(feats0.T, W1)
    s1 = _tc_skip1(skip1, W1)
    g1 = _sc_gather(z2, idx1, N1P, 128, 7, 56, table_rows=N2P)
    y = _tc_mid(g1, s1, gamma1, beta1, w2a)
    s0_t = _tc_skip0(skip0, W2, b2)
    g0 = _sc_gather_stream(y, idx0, N0P, 128, 14, 112, nbuf=8)
    return _tc_add(g0, s0_t).T
